# Initial kernel scaffold; baseline (speedup 1.0000x reference)
import jax
import jax.numpy as jnp
from jax.experimental import pallas as pl


def kernel(inputs, W):
    raise NotImplementedError("write your pallas kernel here")


# speedup vs baseline: 1.7162x; 1.7162x over previous
import jax
import jax.numpy as jnp
from jax.experimental import pallas as pl

S_TILE = 256


def _add_body(x_ref, w_ref, o_ref):
    o_ref[...] = x_ref[...] + w_ref[...][None, :, :]


def kernel(inputs, W):
    batch, seq_len, dim = inputs.shape
    grid = (seq_len // S_TILE,)
    return pl.pallas_call(
        _add_body,
        grid=grid,
        in_specs=[
            pl.BlockSpec((batch, S_TILE, dim), lambda i: (0, i, 0)),
            pl.BlockSpec((S_TILE, dim), lambda i: (i, 0)),
        ],
        out_specs=pl.BlockSpec((batch, S_TILE, dim), lambda i: (0, i, 0)),
        out_shape=jax.ShapeDtypeStruct((batch, seq_len, dim), inputs.dtype),
    )(inputs, W)


# speedup vs baseline: 1.7312x; 1.0087x over previous
import jax
import jax.numpy as jnp
from jax.experimental import pallas as pl

S_TILE = 512


def _add_body(x_ref, w_ref, o_ref):
    o_ref[...] = x_ref[...] + w_ref[...][None, :, :]


def kernel(inputs, W):
    batch, seq_len, dim = inputs.shape
    grid = (seq_len // S_TILE,)
    return pl.pallas_call(
        _add_body,
        grid=grid,
        in_specs=[
            pl.BlockSpec((batch, S_TILE, dim), lambda i: (0, i, 0)),
            pl.BlockSpec((S_TILE, dim), lambda i: (i, 0)),
        ],
        out_specs=pl.BlockSpec((batch, S_TILE, dim), lambda i: (0, i, 0)),
        out_shape=jax.ShapeDtypeStruct((batch, seq_len, dim), inputs.dtype),
    )(inputs, W)
